# baseline (speedup 1.0000x reference)
import functools

import jax
import jax.numpy as jnp
from jax import lax
from jax.experimental import pallas as pl
from jax.experimental.pallas import tpu as pltpu
from jax.experimental.pallas import tpu_sc as plsc

_B, _L, _D = 16, 512, 256
_HALF = 128
_LANES = 16


def _sc_mean_body(enc_hbm, out_hbm, slab_v, acc_v):
    wid = lax.axis_index("s") * 2 + lax.axis_index("c")
    b = wid // 2
    doff = (wid % 2) * _HALF
    pltpu.sync_copy(enc_hbm.at[b, :, pl.ds(doff, _HALF)], slab_v)

    nvec = _HALF // _LANES

    def body(l, accs):
        return tuple(
            accs[j] + slab_v[l, pl.ds(j * _LANES, _LANES)] for j in range(nvec)
        )

    accs = lax.fori_loop(
        0, _L, body, tuple(jnp.zeros((_LANES,), jnp.float32) for _ in range(nvec))
    )
    for j in range(nvec):
        acc_v[pl.ds(j * _LANES, _LANES)] = accs[j] * (1.0 / _L)
    pltpu.sync_copy(acc_v, out_hbm.at[b, pl.ds(doff, _HALF)])


def kernel(user_id, event_type, enc_output, user_output, adjacent_matrix):
    B, L, D = enc_output.shape
    mesh = plsc.VectorSubcoreMesh(core_axis_name="c", subcore_axis_name="s")
    sc_mean = functools.partial(
        pl.kernel,
        mesh=mesh,
        out_type=jax.ShapeDtypeStruct((B, D), jnp.float32),
        scratch_types=[
            pltpu.VMEM((L, _HALF), jnp.float32),
            pltpu.VMEM((_HALF,), jnp.float32),
        ],
    )(_sc_mean_body)
    return sc_mean(enc_output)


# speedup vs baseline: 3.9588x; 3.9588x over previous
import jax
import jax.numpy as jnp
from jax.experimental import pallas as pl

_BCHUNK = 4


def _mean_kernel(enc_ref, out_ref):
    x = enc_ref[...]
    out_ref[...] = jnp.sum(x, axis=1, keepdims=True) * (1.0 / x.shape[1])


def kernel(user_id, event_type, enc_output, user_output, adjacent_matrix):
    B, L, D = enc_output.shape
    out = pl.pallas_call(
        _mean_kernel,
        grid=(B // _BCHUNK,),
        in_specs=[pl.BlockSpec((_BCHUNK, L, D), lambda i: (i, 0, 0))],
        out_specs=pl.BlockSpec((_BCHUNK, 1, D), lambda i: (i, 0, 0)),
        out_shape=jax.ShapeDtypeStruct((B, 1, D), enc_output.dtype),
    )(enc_output)
    return out.reshape(B, D)


# speedup vs baseline: 5.7026x; 1.4405x over previous
import jax
import jax.numpy as jnp
from jax.experimental import pallas as pl


_BCHUNK = 8


def _mean_kernel(enc_ref, out_ref):
    x = enc_ref[...]
    out_ref[...] = jnp.sum(x, axis=1) * (1.0 / x.shape[1])


def kernel(user_id, event_type, enc_output, user_output, adjacent_matrix):
    B, L, D = enc_output.shape
    out = pl.pallas_call(
        _mean_kernel,
        grid=(B // _BCHUNK,),
        in_specs=[pl.BlockSpec((_BCHUNK, L, D), lambda i: (i, 0, 0))],
        out_specs=pl.BlockSpec((_BCHUNK, D), lambda i: (i, 0)),
        out_shape=jax.ShapeDtypeStruct((B, D), enc_output.dtype),
    )(enc_output)
    return out
